# baseline (speedup 1.0000x reference)
import jax
import jax.numpy as jnp
from jax import lax
from jax.experimental import pallas as pl
from jax.experimental.pallas import tpu as pltpu
from jax.experimental.pallas import tpu_sc as plsc

N = 128
SCALE = 1.5
B = 1048576
D = 28
NC = 2
NS = 16
NW = NC * NS
BPW = B // NW
C = 1024
NCHUNK = BPW // C
NGRP = C // 16
GPD = 128
ND = C // GPD
NBLK = 3
WB = N * N * N * D // 16

CELL = 2.0 * SCALE / N


def _sc_body(x_hbm, d_hbm, grid_hbm, color_hbm, sigma_hbm,
             xv, dv, idxv, offv, maskv, rows, colv, sigv, sem_g):
    wid = lax.axis_index("s") * NC + lax.axis_index("c")
    iota = lax.iota(jnp.int32, 16)
    r48 = iota * 48

    def per_chunk(ci, _):
        base = wid * BPW + ci * C

        pltpu.sync_copy(x_hbm.at[pl.ds(3 * base, 3 * C)], xv)
        pltpu.sync_copy(d_hbm.at[pl.ds(3 * base, 3 * C)], dv)

        def idx_grp(g, _):
            r3 = 3 * (g * 16 + iota)
            x0 = plsc.load_gather(xv, [r3])
            x1 = plsc.load_gather(xv, [r3 + 1])
            x2 = plsc.load_gather(xv, [r3 + 2])
            i0 = jnp.clip((x0 / CELL + N / 2.0).astype(jnp.int32), 0, N - 1)
            i1 = jnp.clip((x1 / CELL + N / 2.0).astype(jnp.int32), 0, N - 1)
            i2 = jnp.clip((x2 / CELL + N / 2.0).astype(jnp.int32), 0, N - 1)
            fi = (i0 * (N * N) + i1 * N) + i2
            w = fi * D
            j0 = w >> 4
            off = w & 15
            m = ((jnp.abs(x0) < SCALE) & (jnp.abs(x1) < SCALE)
                 & (jnp.abs(x2) < SCALE))
            jrow = g // (GPD // 16)
            jcol = pl.ds((g % (GPD // 16)) * 16, 16)
            idxv[0, jrow, jcol] = j0
            idxv[1, jrow, jcol] = j0 + 1
            idxv[2, jrow, jcol] = j0 + 2
            offv[pl.ds(g * 16, 16)] = off
            maskv[pl.ds(g * 16, 16)] = jnp.where(m, 1.0, 0.0)
            return 0

        lax.fori_loop(0, NGRP, idx_grp, 0, unroll=False)

        copies = []
        for b in range(NBLK):
            for j in range(ND):
                copies.append(pltpu.async_copy(
                    grid_hbm.at[idxv.at[b, j]],
                    rows.at[b, pl.ds(j * GPD, GPD)], sem_g))
        for cp in copies:
            cp.wait()

        def sh_grp(g, _):
            r = g * 16 + iota
            r3 = 3 * r
            mk = maskv[pl.ds(g * 16, 16)]
            off = offv[pl.ds(g * 16, 16)]
            dx = plsc.load_gather(dv, [r3])
            dy = plsc.load_gather(dv, [r3 + 1])
            dz = plsc.load_gather(dv, [r3 + 2])

            def ch(c):
                t = off + c
                return plsc.load_gather(rows, [t >> 4, r, t & 15])

            sigv[pl.ds(g * 16, 16)] = mk * jnp.maximum(ch(0), 0.0)

            b0 = jnp.full((16,), 0.282095, jnp.float32)
            b1 = -0.488603 * dy
            b2 = 0.488603 * dz
            b3 = -0.488603 * dx
            b4 = 1.092548 * (dx * dy)
            b5 = -1.092548 * (dy * dz)
            b6 = 0.315392 * (2.0 * dz * dz - dx * dx - dy * dy)
            b7 = -1.092548 * (dx * dz)
            b8 = 0.546274 * (dx * dx - dy * dy)
            basis = (b0, b1, b2, b3, b4, b5, b6, b7, b8)
            for c in range(3):
                acc = basis[0] * ch(1 + 9 * c)
                for j in range(1, 9):
                    acc = acc + basis[j] * ch(1 + 9 * c + j)
                plsc.store_scatter(colv, [r3 + c], mk * acc)
            return 0

        lax.fori_loop(0, NGRP, sh_grp, 0, unroll=False)

        pltpu.sync_copy(colv, color_hbm.at[pl.ds(3 * base, 3 * C)])
        pltpu.sync_copy(sigv, sigma_hbm.at[pl.ds(base, C)])
        return 0

    lax.fori_loop(0, NCHUNK, per_chunk, 0, unroll=False)


def kernel(x, d, voxel_grid):
    xf = x.reshape(3 * B)
    df = d.reshape(3 * B)
    vg = voxel_grid.reshape(WB, 16)
    mesh = plsc.VectorSubcoreMesh(core_axis_name="c", subcore_axis_name="s")
    kfn = pl.kernel(
        _sc_body,
        mesh=mesh,
        compiler_params=pltpu.CompilerParams(
            needs_layout_passes=False, use_tc_tiling_on_sc=False),
        out_type=(
            jax.ShapeDtypeStruct((3 * B,), jnp.float32),
            jax.ShapeDtypeStruct((B,), jnp.float32),
        ),
        scratch_types=[
            pltpu.VMEM((3 * C,), jnp.float32),
            pltpu.VMEM((3 * C,), jnp.float32),
            pltpu.VMEM((NBLK, ND, GPD), jnp.int32),
            pltpu.VMEM((C,), jnp.int32),
            pltpu.VMEM((C,), jnp.float32),
            pltpu.VMEM((NBLK, C, 16), jnp.float32),
            pltpu.VMEM((3 * C,), jnp.float32),
            pltpu.VMEM((C,), jnp.float32),
            pltpu.SemaphoreType.DMA,
        ],
    )
    colf, sigma = kfn(xf, df, vg)
    return colf.reshape(B, 3), sigma


# speedup vs baseline: 2.6932x; 2.6932x over previous
import jax
import jax.numpy as jnp
from jax import lax
from jax.experimental import pallas as pl
from jax.experimental.pallas import tpu as pltpu
from jax.experimental.pallas import tpu_sc as plsc

N = 128
SCALE = 1.5
B = 1048576
D = 28
DP = 32
NV = N * N * N
NWORDS = NV * D
NC = 2
NS = 16
NW = NC * NS
PLANE = N * N
SLAB = D * PLANE

VT = NV // NW
VC = 1024
NVCH = VT // VC

BPW = B // NW
C = 1024
NCHUNK = BPW // C
NGRP = C // 16
GPD = 128
ND = C // GPD

CELL = 2.0 * SCALE / N


def _repack_body(src_hbm, t_hbm, strips, outb, sem_in):
    wid = lax.axis_index("s") * NC + lax.axis_index("c")
    iota = lax.iota(jnp.int32, 16)
    iota2 = iota * 2

    def per_chunk(ci, _):
        v0 = wid * VT + ci * VC
        d0 = v0 // PLANE
        u0 = v0 - d0 * PLANE
        base = d0 * SLAB + u0
        cps = [pltpu.async_copy(src_hbm.at[pl.ds(base + c * PLANE, VC)],
                                strips.at[c], sem_in)
               for c in range(D)]
        for cp in cps:
            cp.wait()

        def grp(k, _):
            vv = k * 16
            rbase = iota2 + 2 * vv
            for c in range(D):
                vals = strips[c, pl.ds(vv, 16)]
                rows_i = rbase + (c // 16)
                word_i = jnp.full((16,), c % 16, jnp.int32)
                plsc.store_scatter(outb, [rows_i, word_i], vals)
            return 0

        lax.fori_loop(0, VC // 16, grp, 0, unroll=False)
        pltpu.sync_copy(outb, t_hbm.at[pl.ds(2 * v0, 2 * VC)])
        return 0

    lax.fori_loop(0, NVCH, per_chunk, 0, unroll=False)


def _shade_body(x0_h, x1_h, x2_h, d0_h, d1_h, d2_h, t_hbm,
                c0_h, c1_h, c2_h, sg_h,
                xyzv, idxv, maskv, rows, outv, sem_in, sem_g):
    wid = lax.axis_index("s") * NC + lax.axis_index("c")

    def per_chunk(ci, _):
        base = wid * BPW + ci * C

        cps = [pltpu.async_copy(h.at[pl.ds(base, C)], xyzv.at[i], sem_in)
               for i, h in enumerate((x0_h, x1_h, x2_h, d0_h, d1_h, d2_h))]
        for cp in cps:
            cp.wait()

        def idx_grp(g, _):
            s = pl.ds(g * 16, 16)
            ax = xyzv[0, s]
            ay = xyzv[1, s]
            az = xyzv[2, s]
            i0 = jnp.clip((ax / CELL + N / 2.0).astype(jnp.int32), 0, N - 1)
            i1 = jnp.clip((ay / CELL + N / 2.0).astype(jnp.int32), 0, N - 1)
            i2 = jnp.clip((az / CELL + N / 2.0).astype(jnp.int32), 0, N - 1)
            fi2 = ((i0 * (N * N) + i1 * N) + i2) * 2
            m = ((jnp.abs(ax) < SCALE) & (jnp.abs(ay) < SCALE)
                 & (jnp.abs(az) < SCALE))
            jrow = g // (GPD // 16)
            jcol = pl.ds((g % (GPD // 16)) * 16, 16)
            idxv[0, jrow, jcol] = fi2
            idxv[1, jrow, jcol] = fi2 + 1
            maskv[pl.ds(g * 16, 16)] = jnp.where(m, 1.0, 0.0)
            return 0

        lax.fori_loop(0, NGRP, idx_grp, 0, unroll=False)

        copies = []
        for b in range(2):
            for j in range(ND):
                copies.append(pltpu.async_copy(
                    t_hbm.at[idxv.at[b, j]],
                    rows.at[b, pl.ds(j * GPD, GPD)], sem_g))
        for cp in copies:
            cp.wait()

        def sh_grp(g, _):
            s = pl.ds(g * 16, 16)
            r = g * 16 + lax.iota(jnp.int32, 16)
            mk = maskv[s]
            dx = xyzv[3, s]
            dy = xyzv[4, s]
            dz = xyzv[5, s]

            def ch(c):
                blk = jnp.full((16,), c // 16, jnp.int32)
                wrd = jnp.full((16,), c % 16, jnp.int32)
                return plsc.load_gather(rows, [blk, r, wrd])

            outv[3, s] = mk * jnp.maximum(ch(0), 0.0)

            b0 = jnp.full((16,), 0.282095, jnp.float32)
            b1 = -0.488603 * dy
            b2 = 0.488603 * dz
            b3 = -0.488603 * dx
            b4 = 1.092548 * (dx * dy)
            b5 = -1.092548 * (dy * dz)
            b6 = 0.315392 * (2.0 * dz * dz - dx * dx - dy * dy)
            b7 = -1.092548 * (dx * dz)
            b8 = 0.546274 * (dx * dx - dy * dy)
            basis = (b0, b1, b2, b3, b4, b5, b6, b7, b8)
            for cc in range(3):
                acc = basis[0] * ch(1 + 9 * cc)
                for j in range(1, 9):
                    acc = acc + basis[j] * ch(1 + 9 * cc + j)
                outv[cc, s] = mk * acc
            return 0

        lax.fori_loop(0, NGRP, sh_grp, 0, unroll=False)

        for i, h in enumerate((c0_h, c1_h, c2_h, sg_h)):
            pltpu.sync_copy(outv.at[i], h.at[pl.ds(base, C)])
        return 0

    lax.fori_loop(0, NCHUNK, per_chunk, 0, unroll=False)


def kernel(x, d, voxel_grid):
    vgt = jnp.transpose(voxel_grid, (0, 3, 1, 2)).reshape(NWORDS)
    mesh = plsc.VectorSubcoreMesh(core_axis_name="c", subcore_axis_name="s")
    params = pltpu.CompilerParams(
        needs_layout_passes=False, use_tc_tiling_on_sc=False)

    repack = pl.kernel(
        _repack_body,
        mesh=mesh,
        compiler_params=params,
        out_type=jax.ShapeDtypeStruct((2 * NV, 16), jnp.float32),
        scratch_types=[
            pltpu.VMEM((D, VC), jnp.float32),
            pltpu.VMEM((2 * VC, 16), jnp.float32),
            pltpu.SemaphoreType.DMA,
        ],
    )
    table = repack(vgt)

    shade = pl.kernel(
        _shade_body,
        mesh=mesh,
        compiler_params=params,
        out_type=(
            jax.ShapeDtypeStruct((B,), jnp.float32),
            jax.ShapeDtypeStruct((B,), jnp.float32),
            jax.ShapeDtypeStruct((B,), jnp.float32),
            jax.ShapeDtypeStruct((B,), jnp.float32),
        ),
        scratch_types=[
            pltpu.VMEM((6, C), jnp.float32),
            pltpu.VMEM((2, ND, GPD), jnp.int32),
            pltpu.VMEM((C,), jnp.float32),
            pltpu.VMEM((2, C, 16), jnp.float32),
            pltpu.VMEM((4, C), jnp.float32),
            pltpu.SemaphoreType.DMA,
            pltpu.SemaphoreType.DMA,
        ],
    )
    c0, c1, c2, sigma = shade(x[:, 0], x[:, 1], x[:, 2],
                              d[:, 0], d[:, 1], d[:, 2], table)
    color = jnp.stack([c0, c1, c2], axis=1)
    return color, sigma


# speedup vs baseline: 2.6950x; 1.0007x over previous
import jax
import jax.numpy as jnp
from jax import lax
from jax.experimental import pallas as pl
from jax.experimental.pallas import tpu as pltpu
from jax.experimental.pallas import tpu_sc as plsc

N = 128
SCALE = 1.5
B = 1048576
D = 28
DP = 32
NV = N * N * N
NWORDS = NV * D
NC = 2
NS = 16
NW = NC * NS
PLANE = N * N
SLAB = D * PLANE

VT = NV // NW
VC = 1024
NVCH = VT // VC

BPW = B // NW
C = 1024
NCHUNK = BPW // C
NGRP = C // 16
GPD = 128
ND = C // GPD

CELL = 2.0 * SCALE / N


def _repack_body(src_hbm, t_hbm, strips, outb, sem_in):
    wid = lax.axis_index("s") * NC + lax.axis_index("c")
    iota = lax.iota(jnp.int32, 16)
    iota2 = iota * 2

    def per_chunk(ci, _):
        v0 = wid * VT + ci * VC
        d0 = v0 // PLANE
        u0 = v0 - d0 * PLANE
        pltpu.async_copy(
            src_hbm.at[pl.ds(d0 * D, D), pl.ds(u0, VC)], strips,
            sem_in).wait()

        def grp(k, _):
            vv = k * 16
            rbase = iota2 + 2 * vv
            for c in range(D):
                vals = strips[c, pl.ds(vv, 16)]
                rows_i = rbase + (c // 16)
                word_i = jnp.full((16,), c % 16, jnp.int32)
                plsc.store_scatter(outb, [rows_i, word_i], vals)
            return 0

        lax.fori_loop(0, VC // 16, grp, 0, unroll=False)
        pltpu.sync_copy(outb, t_hbm.at[pl.ds(2 * v0, 2 * VC)])
        return 0

    lax.fori_loop(0, NVCH, per_chunk, 0, unroll=False)


def _shade_body(x0_h, x1_h, x2_h, d0_h, d1_h, d2_h, t_hbm,
                c0_h, c1_h, c2_h, sg_h,
                xyzv, idxv, maskv, rows, outv, sem_in, sem_g):
    wid = lax.axis_index("s") * NC + lax.axis_index("c")

    def per_chunk(ci, _):
        base = wid * BPW + ci * C

        cps = [pltpu.async_copy(h.at[pl.ds(base, C)], xyzv.at[i], sem_in)
               for i, h in enumerate((x0_h, x1_h, x2_h, d0_h, d1_h, d2_h))]
        for cp in cps:
            cp.wait()

        def idx_grp(g, _):
            s = pl.ds(g * 16, 16)
            ax = xyzv[0, s]
            ay = xyzv[1, s]
            az = xyzv[2, s]
            i0 = jnp.clip((ax / CELL + N / 2.0).astype(jnp.int32), 0, N - 1)
            i1 = jnp.clip((ay / CELL + N / 2.0).astype(jnp.int32), 0, N - 1)
            i2 = jnp.clip((az / CELL + N / 2.0).astype(jnp.int32), 0, N - 1)
            fi2 = ((i0 * (N * N) + i1 * N) + i2) * 2
            m = ((jnp.abs(ax) < SCALE) & (jnp.abs(ay) < SCALE)
                 & (jnp.abs(az) < SCALE))
            jrow = g // (GPD // 16)
            jcol = pl.ds((g % (GPD // 16)) * 16, 16)
            idxv[0, jrow, jcol] = fi2
            idxv[1, jrow, jcol] = fi2 + 1
            maskv[pl.ds(g * 16, 16)] = jnp.where(m, 1.0, 0.0)
            return 0

        lax.fori_loop(0, NGRP, idx_grp, 0, unroll=False)

        copies = []
        for b in range(2):
            for j in range(ND):
                copies.append(pltpu.async_copy(
                    t_hbm.at[idxv.at[b, j]],
                    rows.at[b, pl.ds(j * GPD, GPD)], sem_g))
        for cp in copies:
            cp.wait()

        def sh_grp(g, _):
            s = pl.ds(g * 16, 16)
            r = g * 16 + lax.iota(jnp.int32, 16)
            mk = maskv[s]
            dx = xyzv[3, s]
            dy = xyzv[4, s]
            dz = xyzv[5, s]

            def ch(c):
                blk = jnp.full((16,), c // 16, jnp.int32)
                wrd = jnp.full((16,), c % 16, jnp.int32)
                return plsc.load_gather(rows, [blk, r, wrd])

            outv[3, s] = mk * jnp.maximum(ch(0), 0.0)

            b0 = jnp.full((16,), 0.282095, jnp.float32)
            b1 = -0.488603 * dy
            b2 = 0.488603 * dz
            b3 = -0.488603 * dx
            b4 = 1.092548 * (dx * dy)
            b5 = -1.092548 * (dy * dz)
            b6 = 0.315392 * (2.0 * dz * dz - dx * dx - dy * dy)
            b7 = -1.092548 * (dx * dz)
            b8 = 0.546274 * (dx * dx - dy * dy)
            basis = (b0, b1, b2, b3, b4, b5, b6, b7, b8)
            for cc in range(3):
                acc = basis[0] * ch(1 + 9 * cc)
                for j in range(1, 9):
                    acc = acc + basis[j] * ch(1 + 9 * cc + j)
                outv[cc, s] = mk * acc
            return 0

        lax.fori_loop(0, NGRP, sh_grp, 0, unroll=False)

        for i, h in enumerate((c0_h, c1_h, c2_h, sg_h)):
            pltpu.sync_copy(outv.at[i], h.at[pl.ds(base, C)])
        return 0

    lax.fori_loop(0, NCHUNK, per_chunk, 0, unroll=False)


def kernel(x, d, voxel_grid):
    vgt = jnp.transpose(voxel_grid, (0, 3, 1, 2)).reshape(N * D, PLANE)
    mesh = plsc.VectorSubcoreMesh(core_axis_name="c", subcore_axis_name="s")
    params = pltpu.CompilerParams(
        needs_layout_passes=False, use_tc_tiling_on_sc=False)

    repack = pl.kernel(
        _repack_body,
        mesh=mesh,
        compiler_params=params,
        out_type=jax.ShapeDtypeStruct((2 * NV, 16), jnp.float32),
        scratch_types=[
            pltpu.VMEM((D, VC), jnp.float32),
            pltpu.VMEM((2 * VC, 16), jnp.float32),
            pltpu.SemaphoreType.DMA,
        ],
    )
    table = repack(vgt)

    shade = pl.kernel(
        _shade_body,
        mesh=mesh,
        compiler_params=params,
        out_type=(
            jax.ShapeDtypeStruct((B,), jnp.float32),
            jax.ShapeDtypeStruct((B,), jnp.float32),
            jax.ShapeDtypeStruct((B,), jnp.float32),
            jax.ShapeDtypeStruct((B,), jnp.float32),
        ),
        scratch_types=[
            pltpu.VMEM((6, C), jnp.float32),
            pltpu.VMEM((2, ND, GPD), jnp.int32),
            pltpu.VMEM((C,), jnp.float32),
            pltpu.VMEM((2, C, 16), jnp.float32),
            pltpu.VMEM((4, C), jnp.float32),
            pltpu.SemaphoreType.DMA,
            pltpu.SemaphoreType.DMA,
        ],
    )
    c0, c1, c2, sigma = shade(x[:, 0], x[:, 1], x[:, 2],
                              d[:, 0], d[:, 1], d[:, 2], table)
    color = jnp.stack([c0, c1, c2], axis=1)
    return color, sigma


# speedup vs baseline: 3.4843x; 1.2929x over previous
import jax
import jax.numpy as jnp
from jax import lax
from jax.experimental import pallas as pl
from jax.experimental.pallas import tpu as pltpu
from jax.experimental.pallas import tpu_sc as plsc

N = 128
SCALE = 1.5
B = 1048576
D = 28
DP = 32
NV = N * N * N
NWORDS = NV * D
NC = 2
NS = 16
NW = NC * NS
PLANE = N * N
SLAB = D * PLANE

VT = NV // NW
VC = 1024
NVCH = VT // VC

BPW = B // NW
C = 1024
NCHUNK = BPW // C
NGRP = C // 16
GPD = 128
ND = C // GPD

CELL = 2.0 * SCALE / N


def _repack_body(src_hbm, t_hbm, strips, outb, sem_in):
    wid = lax.axis_index("s") * NC + lax.axis_index("c")
    iota = lax.iota(jnp.int32, 16)
    iota2 = iota * 2

    def per_chunk(ci, _):
        v0 = wid * VT + ci * VC
        d0 = v0 // PLANE
        u0 = v0 - d0 * PLANE
        pltpu.async_copy(
            src_hbm.at[pl.ds(d0 * D, D), pl.ds(u0, VC)], strips,
            sem_in).wait()

        @plsc.parallel_loop(0, VC // 16, unroll=2)
        def grp(k):
            vv = k * 16
            rbase = iota2 + 2 * vv
            vals = [strips[c, pl.ds(vv, 16)] for c in range(D)]
            for c in range(D):
                rows_i = rbase + (c // 16)
                word_i = jnp.full((16,), c % 16, jnp.int32)
                plsc.store_scatter(outb, [rows_i, word_i], vals[c])
        pltpu.sync_copy(outb, t_hbm.at[pl.ds(2 * v0, 2 * VC)])
        return 0

    lax.fori_loop(0, NVCH, per_chunk, 0, unroll=False)


def _shade_body(x0_h, x1_h, x2_h, d0_h, d1_h, d2_h, t_hbm,
                c0_h, c1_h, c2_h, sg_h,
                xyzv, idxv, maskv, rows, outv, sem_in, sem_g):
    wid = lax.axis_index("s") * NC + lax.axis_index("c")

    def per_chunk(ci, _):
        base = wid * BPW + ci * C

        cps = [pltpu.async_copy(h.at[pl.ds(base, C)], xyzv.at[i], sem_in)
               for i, h in enumerate((x0_h, x1_h, x2_h, d0_h, d1_h, d2_h))]
        for cp in cps:
            cp.wait()

        @plsc.parallel_loop(0, NGRP, unroll=2)
        def idx_grp(g):
            s = pl.ds(g * 16, 16)
            ax = xyzv[0, s]
            ay = xyzv[1, s]
            az = xyzv[2, s]
            i0 = jnp.clip((ax / CELL + N / 2.0).astype(jnp.int32), 0, N - 1)
            i1 = jnp.clip((ay / CELL + N / 2.0).astype(jnp.int32), 0, N - 1)
            i2 = jnp.clip((az / CELL + N / 2.0).astype(jnp.int32), 0, N - 1)
            fi2 = ((i0 * (N * N) + i1 * N) + i2) * 2
            m = ((jnp.abs(ax) < SCALE) & (jnp.abs(ay) < SCALE)
                 & (jnp.abs(az) < SCALE))
            jrow = g // (GPD // 16)
            jcol = pl.ds((g % (GPD // 16)) * 16, 16)
            idxv[0, jrow, jcol] = fi2
            idxv[1, jrow, jcol] = fi2 + 1
            maskv[pl.ds(g * 16, 16)] = jnp.where(m, 1.0, 0.0)

        copies = []
        for b in range(2):
            for j in range(ND):
                copies.append(pltpu.async_copy(
                    t_hbm.at[idxv.at[b, j]],
                    rows.at[b, pl.ds(j * GPD, GPD)], sem_g))
        for cp in copies:
            cp.wait()

        @plsc.parallel_loop(0, NGRP, unroll=2)
        def sh_grp(g):
            s = pl.ds(g * 16, 16)
            r = g * 16 + lax.iota(jnp.int32, 16)
            mk = maskv[s]
            dx = xyzv[3, s]
            dy = xyzv[4, s]
            dz = xyzv[5, s]

            def ch(c):
                blk = jnp.full((16,), c // 16, jnp.int32)
                wrd = jnp.full((16,), c % 16, jnp.int32)
                return plsc.load_gather(rows, [blk, r, wrd])

            outv[3, s] = mk * jnp.maximum(ch(0), 0.0)

            b0 = jnp.full((16,), 0.282095, jnp.float32)
            b1 = -0.488603 * dy
            b2 = 0.488603 * dz
            b3 = -0.488603 * dx
            b4 = 1.092548 * (dx * dy)
            b5 = -1.092548 * (dy * dz)
            b6 = 0.315392 * (2.0 * dz * dz - dx * dx - dy * dy)
            b7 = -1.092548 * (dx * dz)
            b8 = 0.546274 * (dx * dx - dy * dy)
            basis = (b0, b1, b2, b3, b4, b5, b6, b7, b8)
            for cc in range(3):
                acc = basis[0] * ch(1 + 9 * cc)
                for j in range(1, 9):
                    acc = acc + basis[j] * ch(1 + 9 * cc + j)
                outv[cc, s] = mk * acc

        for i, h in enumerate((c0_h, c1_h, c2_h, sg_h)):
            pltpu.sync_copy(outv.at[i], h.at[pl.ds(base, C)])
        return 0

    lax.fori_loop(0, NCHUNK, per_chunk, 0, unroll=False)


def kernel(x, d, voxel_grid):
    vgt = jnp.transpose(voxel_grid, (0, 3, 1, 2)).reshape(N * D, PLANE)
    mesh = plsc.VectorSubcoreMesh(core_axis_name="c", subcore_axis_name="s")
    params = pltpu.CompilerParams(
        needs_layout_passes=False, use_tc_tiling_on_sc=False)

    repack = pl.kernel(
        _repack_body,
        mesh=mesh,
        compiler_params=params,
        out_type=jax.ShapeDtypeStruct((2 * NV, 16), jnp.float32),
        scratch_types=[
            pltpu.VMEM((D, VC), jnp.float32),
            pltpu.VMEM((2 * VC, 16), jnp.float32),
            pltpu.SemaphoreType.DMA,
        ],
    )
    table = repack(vgt)

    shade = pl.kernel(
        _shade_body,
        mesh=mesh,
        compiler_params=params,
        out_type=(
            jax.ShapeDtypeStruct((B,), jnp.float32),
            jax.ShapeDtypeStruct((B,), jnp.float32),
            jax.ShapeDtypeStruct((B,), jnp.float32),
            jax.ShapeDtypeStruct((B,), jnp.float32),
        ),
        scratch_types=[
            pltpu.VMEM((6, C), jnp.float32),
            pltpu.VMEM((2, ND, GPD), jnp.int32),
            pltpu.VMEM((C,), jnp.float32),
            pltpu.VMEM((2, C, 16), jnp.float32),
            pltpu.VMEM((4, C), jnp.float32),
            pltpu.SemaphoreType.DMA,
            pltpu.SemaphoreType.DMA,
        ],
    )
    c0, c1, c2, sigma = shade(x[:, 0], x[:, 1], x[:, 2],
                              d[:, 0], d[:, 1], d[:, 2], table)
    color = jnp.stack([c0, c1, c2], axis=1)
    return color, sigma


# speedup vs baseline: 4.1351x; 1.1868x over previous
import jax
import jax.numpy as jnp
from jax import lax
from jax.experimental import pallas as pl
from jax.experimental.pallas import tpu as pltpu
from jax.experimental.pallas import tpu_sc as plsc

N = 128
SCALE = 1.5
B = 1048576
D = 28
DP = 32
NV = N * N * N
NWORDS = NV * D
NC = 2
NS = 16
NW = NC * NS
PLANE = N * N
SLAB = D * PLANE

VT = NV // NW
VC = 1024
NVCH = VT // VC

BPW = B // NW
C = 1024
NCHUNK = BPW // C
NGRP = C // 16
GPD = 128
ND = C // GPD

CELL = 2.0 * SCALE / N


def _repack_body(src_hbm, t_hbm, strips, outb, sem_in, sem_out):
    wid = lax.axis_index("s") * NC + lax.axis_index("c")
    iota = lax.iota(jnp.int32, 16)
    iota2 = iota * 2

    def in_copy(ci, p):
        v0 = wid * VT + ci * VC
        d0 = v0 // PLANE
        u0 = v0 - d0 * PLANE
        return pltpu.make_async_copy(
            src_hbm.at[pl.ds(d0 * D, D), pl.ds(u0, VC)], strips.at[p],
            sem_in)

    def out_copy(ci, p):
        v0 = wid * VT + ci * VC
        return pltpu.make_async_copy(
            outb.at[p], t_hbm.at[pl.ds(2 * v0, 2 * VC)], sem_out)

    def compute(p):
        @plsc.parallel_loop(0, VC // 16, unroll=2)
        def grp(k):
            vv = k * 16
            rbase = iota2 + 2 * vv
            vals = [strips[p, c, pl.ds(vv, 16)] for c in range(D)]
            for c in range(D):
                rows_i = rbase + (c // 16)
                word_i = jnp.full((16,), c % 16, jnp.int32)
                plsc.store_scatter(outb.at[p], [rows_i, word_i], vals[c])

    in_copy(0, 0).start()

    def outer(kk, _):
        for p in (0, 1):
            ci = kk * 2 + p

            @pl.when(ci + 1 < NVCH)
            def _():
                in_copy(ci + 1, 1 - p).start()

            in_copy(ci, p).wait()

            @pl.when(ci >= 2)
            def _():
                out_copy(ci - 2, p).wait()

            compute(p)
            out_copy(ci, p).start()
        return 0

    lax.fori_loop(0, NVCH // 2, outer, 0, unroll=False)
    out_copy(NVCH - 2, 0).wait()
    out_copy(NVCH - 1, 1).wait()


def _shade_body(x0_h, x1_h, x2_h, d0_h, d1_h, d2_h, t_hbm,
                c0_h, c1_h, c2_h, sg_h,
                xyzv, idxv, maskv, rows, outv, sem_in, sem_g):
    wid = lax.axis_index("s") * NC + lax.axis_index("c")

    def per_chunk(ci, _):
        base = wid * BPW + ci * C

        cps = [pltpu.async_copy(h.at[pl.ds(base, C)], xyzv.at[i], sem_in)
               for i, h in enumerate((x0_h, x1_h, x2_h, d0_h, d1_h, d2_h))]
        for cp in cps:
            cp.wait()

        @plsc.parallel_loop(0, NGRP, unroll=2)
        def idx_grp(g):
            s = pl.ds(g * 16, 16)
            ax = xyzv[0, s]
            ay = xyzv[1, s]
            az = xyzv[2, s]
            i0 = jnp.clip((ax / CELL + N / 2.0).astype(jnp.int32), 0, N - 1)
            i1 = jnp.clip((ay / CELL + N / 2.0).astype(jnp.int32), 0, N - 1)
            i2 = jnp.clip((az / CELL + N / 2.0).astype(jnp.int32), 0, N - 1)
            fi2 = ((i0 * (N * N) + i1 * N) + i2) * 2
            m = ((jnp.abs(ax) < SCALE) & (jnp.abs(ay) < SCALE)
                 & (jnp.abs(az) < SCALE))
            jrow = g // (GPD // 16)
            jcol = pl.ds((g % (GPD // 16)) * 16, 16)
            idxv[0, jrow, jcol] = fi2
            idxv[1, jrow, jcol] = fi2 + 1
            maskv[pl.ds(g * 16, 16)] = jnp.where(m, 1.0, 0.0)

        copies = []
        for b in range(2):
            for j in range(ND):
                copies.append(pltpu.async_copy(
                    t_hbm.at[idxv.at[b, j]],
                    rows.at[b, pl.ds(j * GPD, GPD)], sem_g))
        for cp in copies:
            cp.wait()

        @plsc.parallel_loop(0, NGRP, unroll=2)
        def sh_grp(g):
            s = pl.ds(g * 16, 16)
            r = g * 16 + lax.iota(jnp.int32, 16)
            mk = maskv[s]
            dx = xyzv[3, s]
            dy = xyzv[4, s]
            dz = xyzv[5, s]

            def ch(c):
                blk = jnp.full((16,), c // 16, jnp.int32)
                wrd = jnp.full((16,), c % 16, jnp.int32)
                return plsc.load_gather(rows, [blk, r, wrd])

            outv[3, s] = mk * jnp.maximum(ch(0), 0.0)

            b0 = jnp.full((16,), 0.282095, jnp.float32)
            b1 = -0.488603 * dy
            b2 = 0.488603 * dz
            b3 = -0.488603 * dx
            b4 = 1.092548 * (dx * dy)
            b5 = -1.092548 * (dy * dz)
            b6 = 0.315392 * (2.0 * dz * dz - dx * dx - dy * dy)
            b7 = -1.092548 * (dx * dz)
            b8 = 0.546274 * (dx * dx - dy * dy)
            basis = (b0, b1, b2, b3, b4, b5, b6, b7, b8)
            for cc in range(3):
                acc = basis[0] * ch(1 + 9 * cc)
                for j in range(1, 9):
                    acc = acc + basis[j] * ch(1 + 9 * cc + j)
                outv[cc, s] = mk * acc

        for i, h in enumerate((c0_h, c1_h, c2_h, sg_h)):
            pltpu.sync_copy(outv.at[i], h.at[pl.ds(base, C)])
        return 0

    lax.fori_loop(0, NCHUNK, per_chunk, 0, unroll=False)


def kernel(x, d, voxel_grid):
    vgt = jnp.transpose(voxel_grid, (0, 3, 1, 2)).reshape(N * D, PLANE)
    mesh = plsc.VectorSubcoreMesh(core_axis_name="c", subcore_axis_name="s")
    params = pltpu.CompilerParams(
        needs_layout_passes=False, use_tc_tiling_on_sc=False)

    repack = pl.kernel(
        _repack_body,
        mesh=mesh,
        compiler_params=params,
        out_type=jax.ShapeDtypeStruct((2 * NV, 16), jnp.float32),
        scratch_types=[
            pltpu.VMEM((2, D, VC), jnp.float32),
            pltpu.VMEM((2, 2 * VC, 16), jnp.float32),
            pltpu.SemaphoreType.DMA,
            pltpu.SemaphoreType.DMA,
        ],
    )
    table = repack(vgt)

    shade = pl.kernel(
        _shade_body,
        mesh=mesh,
        compiler_params=params,
        out_type=(
            jax.ShapeDtypeStruct((B,), jnp.float32),
            jax.ShapeDtypeStruct((B,), jnp.float32),
            jax.ShapeDtypeStruct((B,), jnp.float32),
            jax.ShapeDtypeStruct((B,), jnp.float32),
        ),
        scratch_types=[
            pltpu.VMEM((6, C), jnp.float32),
            pltpu.VMEM((2, ND, GPD), jnp.int32),
            pltpu.VMEM((C,), jnp.float32),
            pltpu.VMEM((2, C, 16), jnp.float32),
            pltpu.VMEM((4, C), jnp.float32),
            pltpu.SemaphoreType.DMA,
            pltpu.SemaphoreType.DMA,
        ],
    )
    c0, c1, c2, sigma = shade(x[:, 0], x[:, 1], x[:, 2],
                              d[:, 0], d[:, 1], d[:, 2], table)
    color = jnp.stack([c0, c1, c2], axis=1)
    return color, sigma


# speedup vs baseline: 4.1360x; 1.0002x over previous
import jax
import jax.numpy as jnp
from jax import lax
from jax.experimental import pallas as pl
from jax.experimental.pallas import tpu as pltpu
from jax.experimental.pallas import tpu_sc as plsc

N = 128
SCALE = 1.5
B = 1048576
D = 28
DP = 32
NV = N * N * N
NWORDS = NV * D
NC = 2
NS = 16
NW = NC * NS
PLANE = N * N
SLAB = D * PLANE

VT = NV // NW
VC = 1024
NVCH = VT // VC

BPW = B // NW
C = 1024
NCHUNK = BPW // C
NGRP = C // 16
GPD = 128
ND = C // GPD

CELL = 2.0 * SCALE / N


def _repack_body(src_hbm, t_hbm, strips, outb, sem_in, sem_out):
    wid = lax.axis_index("s") * NC + lax.axis_index("c")
    iota = lax.iota(jnp.int32, 16)
    iota2 = iota * 2

    def in_copy(ci, p):
        v0 = wid * VT + ci * VC
        d0 = v0 // PLANE
        u0 = v0 - d0 * PLANE
        return pltpu.make_async_copy(
            src_hbm.at[pl.ds(d0 * D, D), pl.ds(u0, VC)], strips.at[p],
            sem_in)

    def out_copy(ci, p):
        v0 = wid * VT + ci * VC
        return pltpu.make_async_copy(
            outb.at[p], t_hbm.at[pl.ds(2 * v0, 2 * VC)], sem_out)

    def compute(p):
        @plsc.parallel_loop(0, VC // 16, unroll=2)
        def grp(k):
            vv = k * 16
            rbase = iota2 + 2 * vv
            vals = [strips[p, cc, pl.ds(vv, 16)] for cc in range(D)]
            for cc in range(D):
                rows_i = rbase + (cc // 16)
                word_i = jnp.full((16,), cc % 16, jnp.int32)
                plsc.store_scatter(outb.at[p], [rows_i, word_i], vals[cc])

    in_copy(0, 0).start()

    def outer(kk, _):
        for p in (0, 1):
            ci = kk * 2 + p

            @pl.when(ci + 1 < NVCH)
            def _():
                in_copy(ci + 1, 1 - p).start()

            in_copy(ci, p).wait()

            @pl.when(ci >= 2)
            def _():
                out_copy(ci - 2, p).wait()

            compute(p)
            out_copy(ci, p).start()
        return 0

    lax.fori_loop(0, NVCH // 2, outer, 0, unroll=False)
    out_copy(NVCH - 2, 0).wait()
    out_copy(NVCH - 1, 1).wait()


def _shade_body(x0_h, x1_h, x2_h, d0_h, d1_h, d2_h, t_hbm,
                c0_h, c1_h, c2_h, sg_h,
                xyzv, idxv, maskv, rows, outv, sem_in, sem_g):
    wid = lax.axis_index("s") * NC + lax.axis_index("c")

    def per_chunk(ci, _):
        base = wid * BPW + ci * C

        cps = [pltpu.async_copy(h.at[pl.ds(base, C)], xyzv.at[i], sem_in)
               for i, h in enumerate((x0_h, x1_h, x2_h, d0_h, d1_h, d2_h))]
        for cp in cps:
            cp.wait()

        @plsc.parallel_loop(0, NGRP, unroll=2)
        def idx_grp(g):
            s = pl.ds(g * 16, 16)
            ax = xyzv[0, s]
            ay = xyzv[1, s]
            az = xyzv[2, s]
            i0 = jnp.clip((ax / CELL + N / 2.0).astype(jnp.int32), 0, N - 1)
            i1 = jnp.clip((ay / CELL + N / 2.0).astype(jnp.int32), 0, N - 1)
            i2 = jnp.clip((az / CELL + N / 2.0).astype(jnp.int32), 0, N - 1)
            fi2 = ((i0 * (N * N) + i1 * N) + i2) * 2
            m = ((jnp.abs(ax) < SCALE) & (jnp.abs(ay) < SCALE)
                 & (jnp.abs(az) < SCALE))
            jrow = g // (GPD // 16)
            jcol = pl.ds((g % (GPD // 16)) * 16, 16)
            idxv[0, jrow, jcol] = fi2
            idxv[1, jrow, jcol] = fi2 + 1
            maskv[pl.ds(g * 16, 16)] = jnp.where(m, 1.0, 0.0)

        copies = []
        for b in range(2):
            for j in range(ND):
                copies.append(pltpu.async_copy(
                    t_hbm.at[idxv.at[b, j]],
                    rows.at[b, pl.ds(j * GPD, GPD)], sem_g))
        for cp in copies:
            cp.wait()

        @plsc.parallel_loop(0, NGRP, unroll=2)
        def sh_grp(g):
            s = pl.ds(g * 16, 16)
            r = g * 16 + lax.iota(jnp.int32, 16)
            mk = maskv[s]
            dx = xyzv[3, s]
            dy = xyzv[4, s]
            dz = xyzv[5, s]

            def ch(c):
                blk = jnp.full((16,), c // 16, jnp.int32)
                wrd = jnp.full((16,), c % 16, jnp.int32)
                return plsc.load_gather(rows, [blk, r, wrd])

            outv[3, s] = mk * jnp.maximum(ch(0), 0.0)

            b0 = jnp.full((16,), 0.282095, jnp.float32)
            b1 = -0.488603 * dy
            b2 = 0.488603 * dz
            b3 = -0.488603 * dx
            b4 = 1.092548 * (dx * dy)
            b5 = -1.092548 * (dy * dz)
            b6 = 0.315392 * (2.0 * dz * dz - dx * dx - dy * dy)
            b7 = -1.092548 * (dx * dz)
            b8 = 0.546274 * (dx * dx - dy * dy)
            basis = (b0, b1, b2, b3, b4, b5, b6, b7, b8)
            for cc in range(3):
                acc = basis[0] * ch(1 + 9 * cc)
                for j in range(1, 9):
                    acc = acc + basis[j] * ch(1 + 9 * cc + j)
                outv[cc, s] = mk * acc

        for i, h in enumerate((c0_h, c1_h, c2_h, sg_h)):
            pltpu.sync_copy(outv.at[i], h.at[pl.ds(base, C)])
        return 0

    lax.fori_loop(0, NCHUNK, per_chunk, 0, unroll=False)


def kernel(x, d, voxel_grid):
    vgt = jnp.transpose(voxel_grid, (0, 3, 1, 2)).reshape(N * D, PLANE)
    mesh = plsc.VectorSubcoreMesh(core_axis_name="c", subcore_axis_name="s")
    params = pltpu.CompilerParams(
        needs_layout_passes=False, use_tc_tiling_on_sc=False)

    repack = pl.kernel(
        _repack_body,
        mesh=mesh,
        compiler_params=params,
        out_type=jax.ShapeDtypeStruct((2 * NV, 16), jnp.float32),
        scratch_types=[
            pltpu.VMEM((2, D, VC), jnp.float32),
            pltpu.VMEM((2, 2 * VC, 16), jnp.float32),
            pltpu.SemaphoreType.DMA,
            pltpu.SemaphoreType.DMA,
        ],
    )
    table = repack(vgt)

    shade = pl.kernel(
        _shade_body,
        mesh=mesh,
        compiler_params=params,
        out_type=(
            jax.ShapeDtypeStruct((B,), jnp.float32),
            jax.ShapeDtypeStruct((B,), jnp.float32),
            jax.ShapeDtypeStruct((B,), jnp.float32),
            jax.ShapeDtypeStruct((B,), jnp.float32),
        ),
        scratch_types=[
            pltpu.VMEM((6, C), jnp.float32),
            pltpu.VMEM((2, ND, GPD), jnp.int32),
            pltpu.VMEM((C,), jnp.float32),
            pltpu.VMEM((2, C, 16), jnp.float32),
            pltpu.VMEM((4, C), jnp.float32),
            pltpu.SemaphoreType.DMA,
            pltpu.SemaphoreType.DMA,
        ],
    )
    c0, c1, c2, sigma = shade(x[:, 0], x[:, 1], x[:, 2],
                              d[:, 0], d[:, 1], d[:, 2], table)
    color = jnp.stack([c0, c1, c2], axis=1)
    return color, sigma


# speedup vs baseline: 12.0019x; 2.9018x over previous
import jax
import jax.numpy as jnp
from jax import lax
from jax.experimental import pallas as pl
from jax.experimental.pallas import tpu as pltpu
from jax.experimental.pallas import tpu_sc as plsc

N = 128
SCALE = 1.5
B = 1048576
D = 28
DP = 32
NV = N * N * N
NWORDS = NV * D
NC = 2
NS = 16
NW = NC * NS
PLANE = N * N
SLAB = D * PLANE

VT = NV // NW
VC = 1024
NVCH = VT // VC

BPW = B // NW
C = 1024
NCHUNK = BPW // C
NGRP = C // 16
GPD = 128
ND = C // GPD

CELL = 2.0 * SCALE / N


def _repack_body(src_hbm, t_hbm, strips, outb, sem_in, sem_out):
    wid = lax.axis_index("s") * NC + lax.axis_index("c")
    iota = lax.iota(jnp.int32, 16)
    iota2 = iota * 2

    def in_copy(ci, p):
        v0 = wid * VT + ci * VC
        d0 = v0 // PLANE
        u0 = v0 - d0 * PLANE
        return pltpu.make_async_copy(
            src_hbm.at[pl.ds(d0 * D, D), pl.ds(u0, VC)], strips.at[p],
            sem_in)

    def out_copy(ci, p):
        v0 = wid * VT + ci * VC
        return pltpu.make_async_copy(
            outb.at[p], t_hbm.at[pl.ds(v0, VC)], sem_out)

    def compute(p):
        @plsc.parallel_loop(0, VC // 16, unroll=2)
        def grp(k):
            vv = k * 16
            rbase = iota + vv
            vals = [strips[p, cc, pl.ds(vv, 16)] for cc in range(D)]
            for w in range(D // 2):
                pk = plsc.pack(vals[2 * w], vals[2 * w + 1],
                               format=plsc.PackFormat.INTERLEAVED)
                pi = plsc.bitcast(pk, jnp.int32)
                word_i = jnp.full((16,), w, jnp.int32)
                plsc.store_scatter(outb.at[p], [rbase, word_i], pi)

    in_copy(0, 0).start()

    def outer(kk, _):
        for p in (0, 1):
            ci = kk * 2 + p

            @pl.when(ci + 1 < NVCH)
            def _():
                in_copy(ci + 1, 1 - p).start()

            in_copy(ci, p).wait()

            @pl.when(ci >= 2)
            def _():
                out_copy(ci - 2, p).wait()

            compute(p)
            out_copy(ci, p).start()
        return 0

    lax.fori_loop(0, NVCH // 2, outer, 0, unroll=False)
    out_copy(NVCH - 2, 0).wait()
    out_copy(NVCH - 1, 1).wait()


def _shade_body(x0_h, x1_h, x2_h, d0_h, d1_h, d2_h, t_hbm,
                c0_h, c1_h, c2_h, sg_h,
                xyzv, idxv, maskv, rows, outv, sem_in, sem_g):
    wid = lax.axis_index("s") * NC + lax.axis_index("c")

    def per_chunk(ci, _):
        base = wid * BPW + ci * C

        cps = [pltpu.async_copy(h.at[pl.ds(base, C)], xyzv.at[i], sem_in)
               for i, h in enumerate((x0_h, x1_h, x2_h, d0_h, d1_h, d2_h))]
        for cp in cps:
            cp.wait()

        @plsc.parallel_loop(0, NGRP, unroll=2)
        def idx_grp(g):
            s = pl.ds(g * 16, 16)
            ax = xyzv[0, s]
            ay = xyzv[1, s]
            az = xyzv[2, s]
            i0 = jnp.clip((ax / CELL + N / 2.0).astype(jnp.int32), 0, N - 1)
            i1 = jnp.clip((ay / CELL + N / 2.0).astype(jnp.int32), 0, N - 1)
            i2 = jnp.clip((az / CELL + N / 2.0).astype(jnp.int32), 0, N - 1)
            fi = (i0 * (N * N) + i1 * N) + i2
            m = ((jnp.abs(ax) < SCALE) & (jnp.abs(ay) < SCALE)
                 & (jnp.abs(az) < SCALE))
            jrow = g // (GPD // 16)
            jcol = pl.ds((g % (GPD // 16)) * 16, 16)
            idxv[jrow, jcol] = fi
            maskv[pl.ds(g * 16, 16)] = jnp.where(m, 1.0, 0.0)

        copies = [pltpu.async_copy(
            t_hbm.at[idxv.at[j]],
            rows.at[pl.ds(j * GPD, GPD)], sem_g) for j in range(ND)]
        for cp in copies:
            cp.wait()

        @plsc.parallel_loop(0, NGRP, unroll=2)
        def sh_grp(g):
            s = pl.ds(g * 16, 16)
            r = g * 16 + lax.iota(jnp.int32, 16)
            mk = maskv[s]
            dx = xyzv[3, s]
            dy = xyzv[4, s]
            dz = xyzv[5, s]

            chs = []
            for w in range(D // 2):
                wv = plsc.load_gather(rows, [r, jnp.full((16,), w, jnp.int32)])
                a, b = plsc.unpack(plsc.bitcast(wv, jnp.bfloat16),
                                   format=plsc.PackFormat.INTERLEAVED)
                chs.append(a)
                chs.append(b)

            def ch(c):
                return chs[c]

            outv[3, s] = mk * jnp.maximum(ch(0), 0.0)

            b0 = jnp.full((16,), 0.282095, jnp.float32)
            b1 = -0.488603 * dy
            b2 = 0.488603 * dz
            b3 = -0.488603 * dx
            b4 = 1.092548 * (dx * dy)
            b5 = -1.092548 * (dy * dz)
            b6 = 0.315392 * (2.0 * dz * dz - dx * dx - dy * dy)
            b7 = -1.092548 * (dx * dz)
            b8 = 0.546274 * (dx * dx - dy * dy)
            basis = (b0, b1, b2, b3, b4, b5, b6, b7, b8)
            for cc in range(3):
                acc = basis[0] * ch(1 + 9 * cc)
                for j in range(1, 9):
                    acc = acc + basis[j] * ch(1 + 9 * cc + j)
                outv[cc, s] = mk * acc

        for i, h in enumerate((c0_h, c1_h, c2_h, sg_h)):
            pltpu.sync_copy(outv.at[i], h.at[pl.ds(base, C)])
        return 0

    lax.fori_loop(0, NCHUNK, per_chunk, 0, unroll=False)


def kernel(x, d, voxel_grid):
    vgt = jnp.transpose(voxel_grid, (0, 3, 1, 2)).reshape(N * D, PLANE)
    mesh = plsc.VectorSubcoreMesh(core_axis_name="c", subcore_axis_name="s")
    params = pltpu.CompilerParams(
        needs_layout_passes=False, use_tc_tiling_on_sc=False)

    repack = pl.kernel(
        _repack_body,
        mesh=mesh,
        compiler_params=params,
        out_type=jax.ShapeDtypeStruct((NV, 16), jnp.int32),
        scratch_types=[
            pltpu.VMEM((2, D, VC), jnp.float32),
            pltpu.VMEM((2, VC, 16), jnp.int32),
            pltpu.SemaphoreType.DMA,
            pltpu.SemaphoreType.DMA,
        ],
    )
    table = repack(vgt)

    shade = pl.kernel(
        _shade_body,
        mesh=mesh,
        compiler_params=params,
        out_type=(
            jax.ShapeDtypeStruct((B,), jnp.float32),
            jax.ShapeDtypeStruct((B,), jnp.float32),
            jax.ShapeDtypeStruct((B,), jnp.float32),
            jax.ShapeDtypeStruct((B,), jnp.float32),
        ),
        scratch_types=[
            pltpu.VMEM((6, C), jnp.float32),
            pltpu.VMEM((ND, GPD), jnp.int32),
            pltpu.VMEM((C,), jnp.float32),
            pltpu.VMEM((C, 16), jnp.int32),
            pltpu.VMEM((4, C), jnp.float32),
            pltpu.SemaphoreType.DMA,
            pltpu.SemaphoreType.DMA,
        ],
    )
    c0, c1, c2, sigma = shade(x[:, 0], x[:, 1], x[:, 2],
                              d[:, 0], d[:, 1], d[:, 2], table)
    color = jnp.stack([c0, c1, c2], axis=1)
    return color, sigma


# speedup vs baseline: 12.0204x; 1.0015x over previous
import jax
import jax.numpy as jnp
from jax import lax
from jax.experimental import pallas as pl
from jax.experimental.pallas import tpu as pltpu
from jax.experimental.pallas import tpu_sc as plsc

N = 128
SCALE = 1.5
B = 1048576
D = 28
NV = N * N * N
NC = 2
NS = 16
NW = NC * NS
PLANE = N * N

VT = NV // NW
VC = 1024
NVCH = VT // VC

BPW = B // NW
C = 1024
NCHUNK = BPW // C
NGRP = C // 16
GPD = 128
ND = C // GPD

CELL = 2.0 * SCALE / N


def _repack_body(src_hbm, t_hbm, strips, outb, sem_in, sem_out):
    wid = lax.axis_index("s") * NC + lax.axis_index("c")
    iota = lax.iota(jnp.int32, 16)

    def in_copy(ci, p):
        v0 = wid * VT + ci * VC
        d0 = v0 // PLANE
        u0 = v0 - d0 * PLANE
        return pltpu.make_async_copy(
            src_hbm.at[pl.ds(d0 * D, D), pl.ds(u0, VC)], strips.at[p],
            sem_in)

    def out_copy(ci, p):
        v0 = wid * VT + ci * VC
        return pltpu.make_async_copy(
            outb.at[p], t_hbm.at[pl.ds(v0, VC)], sem_out)

    def compute(p):
        @plsc.parallel_loop(0, VC // 16, unroll=2)
        def grp(k):
            vv = k * 16
            rbase = iota + vv
            vals = [strips[p, cc, pl.ds(vv, 16)] for cc in range(D)]
            for w in range(D // 2):
                pk = plsc.pack(vals[2 * w], vals[2 * w + 1],
                               format=plsc.PackFormat.INTERLEAVED)
                pi = plsc.bitcast(pk, jnp.int32)
                word_i = jnp.full((16,), w, jnp.int32)
                plsc.store_scatter(outb.at[p], [rbase, word_i], pi)

    in_copy(0, 0).start()

    def outer(kk, _):
        for p in (0, 1):
            ci = kk * 2 + p

            @pl.when(ci + 1 < NVCH)
            def _():
                in_copy(ci + 1, 1 - p).start()

            in_copy(ci, p).wait()

            @pl.when(ci >= 2)
            def _():
                out_copy(ci - 2, p).wait()

            compute(p)
            out_copy(ci, p).start()
        return 0

    lax.fori_loop(0, NVCH // 2, outer, 0, unroll=False)
    out_copy(NVCH - 2, 0).wait()
    out_copy(NVCH - 1, 1).wait()


def _shade_body(x0_h, x1_h, x2_h, d0_h, d1_h, d2_h, t_hbm,
                c0_h, c1_h, c2_h, sg_h,
                xyzv, idxv, maskv, rows, outv, sem_in, sem_g):
    wid = lax.axis_index("s") * NC + lax.axis_index("c")

    def per_chunk(ci, _):
        base = wid * BPW + ci * C

        cps = [pltpu.async_copy(h.at[pl.ds(base, C)], xyzv.at[i], sem_in)
               for i, h in enumerate((x0_h, x1_h, x2_h, d0_h, d1_h, d2_h))]
        for cp in cps:
            cp.wait()

        @plsc.parallel_loop(0, NGRP, unroll=2)
        def idx_grp(g):
            s = pl.ds(g * 16, 16)
            ax = xyzv[0, s]
            ay = xyzv[1, s]
            az = xyzv[2, s]
            i0 = jnp.clip((ax / CELL + N / 2.0).astype(jnp.int32), 0, N - 1)
            i1 = jnp.clip((ay / CELL + N / 2.0).astype(jnp.int32), 0, N - 1)
            i2 = jnp.clip((az / CELL + N / 2.0).astype(jnp.int32), 0, N - 1)
            fi = (i0 * (N * N) + i1 * N) + i2
            m = ((jnp.abs(ax) < SCALE) & (jnp.abs(ay) < SCALE)
                 & (jnp.abs(az) < SCALE))
            jrow = g // (GPD // 16)
            jcol = pl.ds((g % (GPD // 16)) * 16, 16)
            idxv[jrow, jcol] = fi
            maskv[pl.ds(g * 16, 16)] = jnp.where(m, 1.0, 0.0)

        copies = [pltpu.async_copy(
            t_hbm.at[idxv.at[j]],
            rows.at[pl.ds(j * GPD, GPD)], sem_g) for j in range(ND)]
        for cp in copies:
            cp.wait()

        @plsc.parallel_loop(0, NGRP, unroll=2)
        def sh_grp(g):
            s = pl.ds(g * 16, 16)
            r = g * 16 + lax.iota(jnp.int32, 16)
            mk = maskv[s]
            dx = xyzv[3, s]
            dy = xyzv[4, s]
            dz = xyzv[5, s]

            chs = []
            for w in range(D // 2):
                wv = plsc.load_gather(rows, [r, jnp.full((16,), w, jnp.int32)])
                a, b = plsc.unpack(plsc.bitcast(wv, jnp.bfloat16),
                                   format=plsc.PackFormat.INTERLEAVED)
                chs.append(a)
                chs.append(b)

            def ch(c):
                return chs[c]

            outv[3, s] = mk * jnp.maximum(ch(0), 0.0)

            b0 = jnp.full((16,), 0.282095, jnp.float32)
            b1 = -0.488603 * dy
            b2 = 0.488603 * dz
            b3 = -0.488603 * dx
            b4 = 1.092548 * (dx * dy)
            b5 = -1.092548 * (dy * dz)
            b6 = 0.315392 * (2.0 * dz * dz - dx * dx - dy * dy)
            b7 = -1.092548 * (dx * dz)
            b8 = 0.546274 * (dx * dx - dy * dy)
            basis = (b0, b1, b2, b3, b4, b5, b6, b7, b8)
            for cc in range(3):
                acc = basis[0] * ch(1 + 9 * cc)
                for j in range(1, 9):
                    acc = acc + basis[j] * ch(1 + 9 * cc + j)
                outv[cc, s] = mk * acc

        for i, h in enumerate((c0_h, c1_h, c2_h, sg_h)):
            pltpu.sync_copy(outv.at[i], h.at[pl.ds(base, C)])
        return 0

    lax.fori_loop(0, NCHUNK, per_chunk, 0, unroll=False)


def kernel(x, d, voxel_grid):
    vgt = jnp.transpose(voxel_grid, (0, 3, 1, 2)).reshape(N * D, PLANE)
    mesh = plsc.VectorSubcoreMesh(core_axis_name="c", subcore_axis_name="s")
    params = pltpu.CompilerParams(
        needs_layout_passes=False, use_tc_tiling_on_sc=False)

    repack = pl.kernel(
        _repack_body,
        mesh=mesh,
        compiler_params=params,
        out_type=jax.ShapeDtypeStruct((NV, 16), jnp.int32),
        scratch_types=[
            pltpu.VMEM((2, D, VC), jnp.float32),
            pltpu.VMEM((2, VC, 16), jnp.int32),
            pltpu.SemaphoreType.DMA,
            pltpu.SemaphoreType.DMA,
        ],
    )
    table = repack(vgt)

    shade = pl.kernel(
        _shade_body,
        mesh=mesh,
        compiler_params=params,
        out_type=(
            jax.ShapeDtypeStruct((B,), jnp.float32),
            jax.ShapeDtypeStruct((B,), jnp.float32),
            jax.ShapeDtypeStruct((B,), jnp.float32),
            jax.ShapeDtypeStruct((B,), jnp.float32),
        ),
        scratch_types=[
            pltpu.VMEM((6, C), jnp.float32),
            pltpu.VMEM((ND, GPD), jnp.int32),
            pltpu.VMEM((C,), jnp.float32),
            pltpu.VMEM((C, 16), jnp.int32),
            pltpu.VMEM((4, C), jnp.float32),
            pltpu.SemaphoreType.DMA,
            pltpu.SemaphoreType.DMA,
        ],
    )
    c0, c1, c2, sigma = shade(x[:, 0], x[:, 1], x[:, 2],
                              d[:, 0], d[:, 1], d[:, 2], table)
    color = jnp.stack([c0, c1, c2], axis=1)
    return color, sigma
